# baseline (speedup 1.0000x reference)
import functools

import jax
import jax.numpy as jnp
from jax import lax
from jax.experimental import pallas as pl
from jax.experimental.pallas import tpu as pltpu
from jax.experimental.pallas import tpu_sc as plsc

_L = 16
_UNROLL = 8


@functools.cache
def _build(batch, feat_dim, num_classes):
    info = plsc.get_sparse_core_info()
    nc, ns = info.num_cores, info.num_subcores
    nw = nc * ns
    rows_per_w = feat_dim // nw
    f_chunk = 2048
    n_fchunk = batch // f_chunk
    scale = 0.5 / batch

    mesh = plsc.VectorSubcoreMesh(core_axis_name="c", subcore_axis_name="s")

    @functools.partial(
        pl.kernel,
        out_type=jax.ShapeDtypeStruct((nw, _L), jnp.float32),
        mesh=mesh,
        compiler_params=pltpu.CompilerParams(
            use_tc_tiling_on_sc=True, needs_layout_passes=False),
        scratch_types=[
            pltpu.VMEM((num_classes,), jnp.float32),
            pltpu.VMEM((batch,), jnp.int32),
            pltpu.VMEM((f_chunk,), jnp.float32),
            pltpu.VMEM((f_chunk,), jnp.float32),
            pltpu.VMEM((_L,), jnp.float32),
            pltpu.SemaphoreType.DMA,
            pltpu.SemaphoreType.DMA,
            pltpu.SemaphoreType.DMA,
        ],
    )
    def k(featT_hbm, lab_hbm, centT_hbm, out_hbm, row_v, lab_v, feat_v0,
          feat_v1, acc_v, rsem, fsem0, fsem1):
        wid = lax.axis_index("s") * nc + lax.axis_index("c")

        rcopy = pltpu.async_copy(centT_hbm.at[wid * rows_per_w], row_v, rsem)
        pltpu.sync_copy(lab_hbm, lab_v)

        fbufs = (feat_v0, feat_v1)
        fsems = (fsem0, fsem1)

        acc = (jnp.zeros((_L,), jnp.float32),) * 4
        for r in range(rows_per_w):
            f = wid * rows_per_w + r
            fcopies = [None] * n_fchunk
            fcopies[0] = pltpu.async_copy(
                featT_hbm.at[f, pl.ds(0, f_chunk)], fbufs[0], fsems[0])
            rcopy.wait()
            for h in range(n_fchunk):
                if h + 1 < n_fchunk:
                    b = (h + 1) % 2
                    fcopies[h + 1] = pltpu.async_copy(
                        featT_hbm.at[f, pl.ds((h + 1) * f_chunk, f_chunk)],
                        fbufs[b], fsems[b])
                fcopies[h].wait()
                feat_v = fbufs[h % 2]

                def body(it, acc):
                    base = it * (_L * _UNROLL)
                    acc = list(acc)
                    for u in range(_UNROLL):
                        o = base + u * _L
                        idx = lab_v[pl.ds(h * f_chunk + o, _L)]
                        c = plsc.load_gather(row_v, [idx])
                        fv = feat_v[pl.ds(o, _L)]
                        d = fv - c
                        acc[u % 4] = acc[u % 4] + d * d
                    return tuple(acc)

                acc = lax.fori_loop(0, f_chunk // (_L * _UNROLL), body, acc)
            if r < rows_per_w - 1:
                rcopy = pltpu.async_copy(
                    centT_hbm.at[wid * rows_per_w + r + 1], row_v, rsem)

        acc_v[...] = (acc[0] + acc[1] + (acc[2] + acc[3])) * scale
        pltpu.sync_copy(acc_v, out_hbm.at[wid])

    return k


def kernel(features, labels, centers):
    batch, feat_dim = features.shape
    num_classes = centers.shape[0]
    k = _build(batch, feat_dim, num_classes)
    partials = k(features.T, labels.astype(jnp.int32), centers.T)
    return jnp.sum(partials)


# speedup vs baseline: 1.0819x; 1.0819x over previous
import functools

import jax
import jax.numpy as jnp
from jax import lax
from jax.experimental import pallas as pl
from jax.experimental.pallas import tpu as pltpu
from jax.experimental.pallas import tpu_sc as plsc

_L = 16
_UNROLL = 8


@functools.cache
def _build(batch, feat_dim, num_classes):
    info = plsc.get_sparse_core_info()
    nc, ns = info.num_cores, info.num_subcores
    nw = nc * ns
    rows_per_w = feat_dim // nw
    f_chunk = 4096
    n_fchunk = batch // f_chunk
    scale = 0.5 / batch

    mesh = plsc.VectorSubcoreMesh(core_axis_name="c", subcore_axis_name="s")

    @functools.partial(
        pl.kernel,
        out_type=jax.ShapeDtypeStruct((nw, _L), jnp.float32),
        mesh=mesh,
        compiler_params=pltpu.CompilerParams(
            use_tc_tiling_on_sc=True, needs_layout_passes=False),
        scratch_types=[
            pltpu.VMEM((num_classes,), jnp.float32),
            pltpu.VMEM((batch,), jnp.int32),
            pltpu.VMEM((f_chunk,), jnp.float32),
            pltpu.VMEM((f_chunk,), jnp.float32),
            pltpu.VMEM((_L,), jnp.float32),
            pltpu.SemaphoreType.DMA,
            pltpu.SemaphoreType.DMA,
            pltpu.SemaphoreType.DMA,
        ],
    )
    def k(featT_hbm, lab_hbm, centT_hbm, out_hbm, row_v, lab_v, feat_v0,
          feat_v1, acc_v, rsem, fsem0, fsem1):
        wid = lax.axis_index("s") * nc + lax.axis_index("c")

        rcopy = pltpu.async_copy(centT_hbm.at[wid * rows_per_w], row_v, rsem)
        pltpu.sync_copy(lab_hbm, lab_v)

        fbufs = (feat_v0, feat_v1)
        fsems = (fsem0, fsem1)

        acc = (jnp.zeros((_L,), jnp.float32),) * 4
        for r in range(rows_per_w):
            f = wid * rows_per_w + r
            fcopies = [None] * n_fchunk
            fcopies[0] = pltpu.async_copy(
                featT_hbm.at[f, pl.ds(0, f_chunk)], fbufs[0], fsems[0])
            rcopy.wait()
            for h in range(n_fchunk):
                if h + 1 < n_fchunk:
                    b = (h + 1) % 2
                    fcopies[h + 1] = pltpu.async_copy(
                        featT_hbm.at[f, pl.ds((h + 1) * f_chunk, f_chunk)],
                        fbufs[b], fsems[b])
                fcopies[h].wait()
                feat_v = fbufs[h % 2]

                def body(it, acc):
                    base = it * (_L * _UNROLL)
                    acc = list(acc)
                    for u in range(_UNROLL):
                        o = base + u * _L
                        idx = lab_v[pl.ds(h * f_chunk + o, _L)]
                        c = plsc.load_gather(row_v, [idx])
                        fv = feat_v[pl.ds(o, _L)]
                        d = fv - c
                        acc[u % 4] = acc[u % 4] + d * d
                    return tuple(acc)

                acc = lax.fori_loop(0, f_chunk // (_L * _UNROLL), body, acc)
            if r < rows_per_w - 1:
                rcopy = pltpu.async_copy(
                    centT_hbm.at[wid * rows_per_w + r + 1], row_v, rsem)

        acc_v[...] = (acc[0] + acc[1] + (acc[2] + acc[3])) * scale
        pltpu.sync_copy(acc_v, out_hbm.at[wid])

    return k


def kernel(features, labels, centers):
    batch, feat_dim = features.shape
    num_classes = centers.shape[0]
    k = _build(batch, feat_dim, num_classes)
    partials = k(features.T, labels.astype(jnp.int32), centers.T)
    return jnp.sum(partials)
